# baseline (speedup 1.0000x reference)
import functools

import jax
import jax.numpy as jnp
from jax.experimental import pallas as pl
from jax.experimental.pallas import tpu as pltpu


def _round_up(v, m):
    return ((v + m - 1) // m) * m


def _fused_kernel(x_ref, w_ref, b_ref, g_ref, beta_ref, o_ref, *, eps, true_out_dim):
    xb = x_ref[...].astype(jnp.bfloat16)
    y = jnp.dot(xb, w_ref[...], preferred_element_type=jnp.float32)
    y = y + b_ref[...]
    inv_d = 1.0 / float(true_out_dim)
    s1 = jnp.sum(y, axis=-1, keepdims=True)
    s2 = jnp.sum(y * y, axis=-1, keepdims=True)
    mean = s1 * inv_d
    var = jnp.maximum(s2 * inv_d - mean * mean, 0.0)
    y = (y - mean) * jax.lax.rsqrt(var + eps)
    y = y * g_ref[...] + beta_ref[...]
    o_ref[...] = jnp.maximum(y, 0.0).astype(o_ref.dtype)


def kernel(x, w, b, gamma, beta, *, eps=1e-5):
    n, in_dim = x.shape
    out_dim = w.shape[1]

    in_pad = _round_up(in_dim, 128)
    out_pad = _round_up(out_dim, 128)
    tm = min(2048, _round_up(n, 8))
    n_pad = _round_up(n, tm)

    xp = x
    if (n_pad, in_pad) != x.shape:
        xp = jnp.zeros((n_pad, in_pad), x.dtype).at[:n, :in_dim].set(x)
    wb = w.astype(jnp.bfloat16)
    if (in_pad, out_pad) != w.shape:
        wb = jnp.zeros((in_pad, out_pad), jnp.bfloat16).at[:in_dim, :out_dim].set(wb)
    bp = b.astype(jnp.float32)
    gp = gamma.astype(jnp.float32)
    betap = beta.astype(jnp.float32)
    if out_pad != out_dim:
        bp = jnp.zeros((1, out_pad), jnp.float32).at[:, :out_dim].set(bp)
        gp = jnp.ones((1, out_pad), jnp.float32).at[:, :out_dim].set(gp)
        betap = jnp.zeros((1, out_pad), jnp.float32).at[:, :out_dim].set(betap)

    body = functools.partial(_fused_kernel, eps=eps, true_out_dim=out_dim)
    y = pl.pallas_call(
        body,
        out_shape=jax.ShapeDtypeStruct((n_pad, out_pad), x.dtype),
        grid=(n_pad // tm,),
        in_specs=[
            pl.BlockSpec((tm, in_pad), lambda m: (m, 0)),
            pl.BlockSpec((in_pad, out_pad), lambda m: (0, 0)),
            pl.BlockSpec((1, out_pad), lambda m: (0, 0)),
            pl.BlockSpec((1, out_pad), lambda m: (0, 0)),
            pl.BlockSpec((1, out_pad), lambda m: (0, 0)),
        ],
        out_specs=pl.BlockSpec((tm, out_pad), lambda m: (m, 0)),
        compiler_params=pltpu.CompilerParams(
            dimension_semantics=("parallel",),
            vmem_limit_bytes=64 * 1024 * 1024,
        ),
    )(xp, wb, bp, gp, betap)

    if (n_pad, out_pad) != (n, out_dim):
        y = y[:n, :out_dim]
    return y


# speedup vs baseline: 1.1467x; 1.1467x over previous
import functools

import jax
import jax.numpy as jnp
from jax.experimental import pallas as pl
from jax.experimental.pallas import tpu as pltpu


def _round_up(v, m):
    return ((v + m - 1) // m) * m


def _fused_kernel(x_ref, w_ref, b_ref, g_ref, beta_ref, o_ref, *, eps, true_out_dim):
    xb = x_ref[...].astype(jnp.bfloat16)
    wb = w_ref[...].astype(jnp.bfloat16)
    y = jnp.dot(xb, wb, preferred_element_type=jnp.float32)
    y = y + b_ref[...]
    inv_d = 1.0 / float(true_out_dim)
    s1 = jnp.sum(y, axis=-1, keepdims=True)
    s2 = jnp.sum(y * y, axis=-1, keepdims=True)
    mean = s1 * inv_d
    var = jnp.maximum(s2 * inv_d - mean * mean, 0.0)
    y = (y - mean) * jax.lax.rsqrt(var + eps)
    y = y * g_ref[...] + beta_ref[...]
    o_ref[...] = jnp.maximum(y, 0.0).astype(o_ref.dtype)


def kernel(x, w, b, gamma, beta, *, eps=1e-5):
    n, in_dim = x.shape
    out_dim = w.shape[1]

    in_pad = _round_up(in_dim, 128)
    out_pad = _round_up(out_dim, 128)
    tm = min(1024, _round_up(n, 8))
    n_pad = _round_up(n, tm)

    xp = x
    if (n_pad, in_pad) != x.shape:
        xp = jnp.zeros((n_pad, in_pad), x.dtype).at[:n, :in_dim].set(x)
    wb = w
    if (in_pad, out_pad) != w.shape:
        wb = jnp.zeros((in_pad, out_pad), w.dtype).at[:in_dim, :out_dim].set(w)
    bp = b.astype(jnp.float32)
    gp = gamma.astype(jnp.float32)
    betap = beta.astype(jnp.float32)
    if out_pad != out_dim:
        bp = jnp.zeros((1, out_pad), jnp.float32).at[:, :out_dim].set(bp)
        gp = jnp.ones((1, out_pad), jnp.float32).at[:, :out_dim].set(gp)
        betap = jnp.zeros((1, out_pad), jnp.float32).at[:, :out_dim].set(betap)

    body = functools.partial(_fused_kernel, eps=eps, true_out_dim=out_dim)
    y = pl.pallas_call(
        body,
        out_shape=jax.ShapeDtypeStruct((n_pad, out_pad), x.dtype),
        grid=(n_pad // tm,),
        in_specs=[
            pl.BlockSpec((tm, in_pad), lambda m: (m, 0)),
            pl.BlockSpec((in_pad, out_pad), lambda m: (0, 0)),
            pl.BlockSpec((1, out_pad), lambda m: (0, 0)),
            pl.BlockSpec((1, out_pad), lambda m: (0, 0)),
            pl.BlockSpec((1, out_pad), lambda m: (0, 0)),
        ],
        out_specs=pl.BlockSpec((tm, out_pad), lambda m: (m, 0)),
        compiler_params=pltpu.CompilerParams(
            dimension_semantics=("parallel",),
            vmem_limit_bytes=64 * 1024 * 1024,
        ),
    )(xp, wb, bp, gp, betap)

    if (n_pad, out_pad) != (n, out_dim):
        y = y[:n, :out_dim]
    return y
